# Initial kernel scaffold; baseline (speedup 1.0000x reference)
import jax
import jax.numpy as jnp
from jax.experimental import pallas as pl


def kernel(x, emb, W, b):
    raise NotImplementedError("write your pallas kernel here")


# speedup vs baseline: 37.3759x; 37.3759x over previous
import functools

import jax
import jax.numpy as jnp
from jax import lax
from jax.experimental import pallas as pl
from jax.experimental.pallas import tpu as pltpu
from jax.experimental.pallas import tpu_sc as plsc

B = 16384
L = 200
V = 1000
D = 64

_NC, _NS = 2, 16
_NW = _NC * _NS
_S = B // _NW
_G = 64
_NCHUNK = _S // _G
_NGRP = L // 16


def _hist_body(x_hbm, c_hbm, idx_v, cnt_v):
    wid = lax.axis_index("s") * _NC + lax.axis_index("c")
    base = wid * _S

    ones = jnp.full((16,), 1.0, jnp.float32)
    zeros = jnp.full((16,), 0.0, jnp.float32)
    lane = lax.iota(jnp.int32, 16)
    tailmask = lane >= (16 - (L - _NGRP * 16))

    def _clear(k, _):
        cnt_v[pl.ds(k * 16, 16)] = zeros
        return _
    lax.fori_loop(0, _G * V // 16, _clear, None)

    def _chunk(k, _):
        row0 = base + k * _G
        pltpu.sync_copy(x_hbm.at[pl.ds(row0 * L, _G * L)], idx_v)

        def _scatter(i, _c):
            rowbase = i * V
            for j in range(_NGRP):
                idx = idx_v[pl.ds(i * L + j * 16, 16)]
                plsc.addupdate_scatter(cnt_v, [idx + rowbase], ones)
            idx = idx_v[pl.ds(i * L + (L - 16), 16)]
            plsc.addupdate_scatter(cnt_v, [idx + rowbase], ones, mask=tailmask)
            return _c
        lax.fori_loop(0, _G, _scatter, None)

        pltpu.sync_copy(cnt_v, c_hbm.at[pl.ds(row0 * V, _G * V)])

        def _rezero(i, _c):
            rowbase = i * V
            for j in range(_NGRP):
                idx = idx_v[pl.ds(i * L + j * 16, 16)]
                plsc.store_scatter(cnt_v, [idx + rowbase], zeros)
            idx = idx_v[pl.ds(i * L + (L - 16), 16)]
            plsc.store_scatter(cnt_v, [idx + rowbase], zeros)
            return _c
        lax.fori_loop(0, _G, _rezero, None)
        return _
    lax.fori_loop(0, _NCHUNK, _chunk, None)


@functools.cache
def _hist():
    return functools.partial(
        pl.kernel,
        mesh=plsc.VectorSubcoreMesh(core_axis_name="c", subcore_axis_name="s"),
        out_type=jax.ShapeDtypeStruct((B * V,), jnp.float32),
        scratch_types=[
            pltpu.VMEM((_G * L,), jnp.int32),
            pltpu.VMEM((_G * V,), jnp.float32),
        ],
        compiler_params=pltpu.CompilerParams(needs_layout_passes=False),
    )(_hist_body)


_BLK = 2048


def _tc_body(c_ref, emb_ref, w_ref, b_ref, o_ref):
    m = jnp.dot(c_ref[...], emb_ref[...], preferred_element_type=jnp.float32)
    r = jnp.maximum(m * (1.0 / L), 0.0)
    y = lax.dot_general(r, w_ref[...], (((1,), (1,)), ((), ())),
                        preferred_element_type=jnp.float32)
    o_ref[...] = jax.nn.sigmoid(y + b_ref[...])


_tc = pl.pallas_call(
    _tc_body,
    grid=(B // _BLK,),
    in_specs=[
        pl.BlockSpec((_BLK, V), lambda i: (i, 0)),
        pl.BlockSpec((V, D), lambda i: (0, 0)),
        pl.BlockSpec((V, D), lambda i: (0, 0)),
        pl.BlockSpec((1, V), lambda i: (0, 0)),
    ],
    out_specs=pl.BlockSpec((_BLK, V), lambda i: (i, 0)),
    out_shape=jax.ShapeDtypeStruct((B, V), jnp.float32),
)


def kernel(x, emb, W, b):
    counts = _hist()(x.astype(jnp.int32).reshape(-1))
    return _tc(counts.reshape(B, V), emb, W, b.reshape(1, V))


# speedup vs baseline: 46.0156x; 1.2312x over previous
import functools

import jax
import jax.numpy as jnp
from jax import lax
from jax.experimental import pallas as pl
from jax.experimental.pallas import tpu as pltpu
from jax.experimental.pallas import tpu_sc as plsc

B = 16384
L = 200
V = 1000
D = 64

_NC, _NS = 2, 16
_NW = _NC * _NS
_S = B // _NW
_G = 64
_NCHUNK = _S // _G
_NGRP = L // 16


def _hist_body(x_hbm, c_hbm, idx_v, cnt_v):
    wid = lax.axis_index("s") * _NC + lax.axis_index("c")
    base = wid * _S

    ones = jnp.full((16,), 1.0, jnp.float32)
    zeros = jnp.full((16,), 0.0, jnp.float32)
    lane = lax.iota(jnp.int32, 16)
    tailmask = lane >= (16 - (L - _NGRP * 16))

    def _clear(k, _):
        cnt_v[pl.ds(k * 16, 16)] = zeros
        return _
    lax.fori_loop(0, _G * V // 16, _clear, None)

    def _chunk(k, _):
        row0 = base + k * _G
        pltpu.sync_copy(x_hbm.at[pl.ds(row0 * L, _G * L)], idx_v)

        def _scatter(i, _c):
            rowbase = i * V
            idxs = [idx_v[pl.ds(i * L + j * 16, 16)] for j in range(_NGRP)]
            idxs.append(idx_v[pl.ds(i * L + (L - 16), 16)])
            addrs = [ix + rowbase for ix in idxs]
            for j in range(_NGRP):
                plsc.addupdate_scatter(cnt_v, [addrs[j]], ones)
            plsc.addupdate_scatter(cnt_v, [addrs[_NGRP]], ones, mask=tailmask)
            return _c
        lax.fori_loop(0, _G, _scatter, None)

        pltpu.sync_copy(cnt_v, c_hbm.at[pl.ds(row0 * V, _G * V)])

        def _rezero(i, _c):
            rowbase = i * V
            idxs = [idx_v[pl.ds(i * L + j * 16, 16)] for j in range(_NGRP)]
            idxs.append(idx_v[pl.ds(i * L + (L - 16), 16)])
            addrs = [ix + rowbase for ix in idxs]
            for j in range(_NGRP + 1):
                plsc.store_scatter(cnt_v, [addrs[j]], zeros)
            return _c
        lax.fori_loop(0, _G, _rezero, None)
        return _
    lax.fori_loop(0, _NCHUNK, _chunk, None)


@functools.cache
def _hist():
    return functools.partial(
        pl.kernel,
        mesh=plsc.VectorSubcoreMesh(core_axis_name="c", subcore_axis_name="s"),
        out_type=jax.ShapeDtypeStruct((B * V,), jnp.float32),
        scratch_types=[
            pltpu.VMEM((_G * L,), jnp.int32),
            pltpu.VMEM((_G * V,), jnp.float32),
        ],
        compiler_params=pltpu.CompilerParams(needs_layout_passes=False),
    )(_hist_body)


_BLK = 2048


def _tc_body(c_ref, emb_ref, w_ref, b_ref, o_ref):
    m = jnp.dot(c_ref[...], emb_ref[...], preferred_element_type=jnp.float32)
    r = jnp.maximum(m * (1.0 / L), 0.0)
    y = lax.dot_general(r, w_ref[...], (((1,), (1,)), ((), ())),
                        preferred_element_type=jnp.float32)
    o_ref[...] = jax.nn.sigmoid(y + b_ref[...])


_tc = pl.pallas_call(
    _tc_body,
    grid=(B // _BLK,),
    in_specs=[
        pl.BlockSpec((_BLK, V), lambda i: (i, 0)),
        pl.BlockSpec((V, D), lambda i: (0, 0)),
        pl.BlockSpec((V, D), lambda i: (0, 0)),
        pl.BlockSpec((1, V), lambda i: (0, 0)),
    ],
    out_specs=pl.BlockSpec((_BLK, V), lambda i: (i, 0)),
    out_shape=jax.ShapeDtypeStruct((B, V), jnp.float32),
)


def kernel(x, emb, W, b):
    counts = _hist()(x.astype(jnp.int32).reshape(-1))
    return _tc(counts.reshape(B, V), emb, W, b.reshape(1, V))


# speedup vs baseline: 66.0000x; 1.4343x over previous
import functools

import jax
import jax.numpy as jnp
from jax import lax
from jax.experimental import pallas as pl
from jax.experimental.pallas import tpu as pltpu
from jax.experimental.pallas import tpu_sc as plsc

B = 16384
L = 200
V = 1000
D = 64

_NC, _NS = 2, 16
_NW = _NC * _NS
_S = B // _NW
_G = 64
_NCHUNK = _S // _G
_NGRP = L // 16


def _hist_body(x_hbm, c_hbm, idx_v, cnt_v):
    wid = lax.axis_index("s") * _NC + lax.axis_index("c")
    base = wid * _S

    ones = jnp.full((16,), 1.0, jnp.float32)
    zeros = jnp.full((16,), 0.0, jnp.float32)
    lane = lax.iota(jnp.int32, 16)
    tailmask = lane >= (16 - (L - _NGRP * 16))

    def _clear(k, _):
        def _clear_row(c, _r):
            cnt_v[k, pl.ds(c * 16, 16)] = zeros
            return _r
        lax.fori_loop(0, V // 16 + 1, _clear_row, None)
        return _
    lax.fori_loop(0, _G, _clear, None)

    def _chunk(k, _):
        row0 = base + k * _G
        pltpu.sync_copy(x_hbm.at[pl.ds(row0, _G)], idx_v)

        def _scatter(i, _c):
            rowv = jnp.full((16,), i, jnp.int32)
            idxs = [idx_v[i, pl.ds(j * 16, 16)] for j in range(_NGRP)]
            idxs.append(idx_v[i, pl.ds(L - 16, 16)])
            for j in range(_NGRP):
                plsc.addupdate_scatter(cnt_v, [rowv, idxs[j]], ones)
            plsc.addupdate_scatter(cnt_v, [rowv, idxs[_NGRP]], ones,
                                   mask=tailmask)
            return _c
        lax.fori_loop(0, _G, _scatter, None)

        pltpu.sync_copy(cnt_v, c_hbm.at[pl.ds(row0, _G)])

        def _rezero(i, _c):
            rowv = jnp.full((16,), i, jnp.int32)
            idxs = [idx_v[i, pl.ds(j * 16, 16)] for j in range(_NGRP)]
            idxs.append(idx_v[i, pl.ds(L - 16, 16)])
            for j in range(_NGRP + 1):
                plsc.store_scatter(cnt_v, [rowv, idxs[j]], zeros)
            return _c
        lax.fori_loop(0, _G, _rezero, None)
        return _
    lax.fori_loop(0, _NCHUNK, _chunk, None)


@functools.cache
def _hist():
    return functools.partial(
        pl.kernel,
        mesh=plsc.VectorSubcoreMesh(core_axis_name="c", subcore_axis_name="s"),
        out_type=jax.ShapeDtypeStruct((B, V), jnp.float32),
        scratch_types=[
            pltpu.VMEM((_G, L), jnp.int32),
            pltpu.VMEM((_G, V), jnp.float32),
        ],
        compiler_params=pltpu.CompilerParams(needs_layout_passes=False),
    )(_hist_body)


_BLK = 2048


def _tc_body(c_ref, emb_ref, w_ref, b_ref, o_ref):
    m = jnp.dot(c_ref[...], emb_ref[...], preferred_element_type=jnp.float32)
    r = jnp.maximum(m * (1.0 / L), 0.0)
    y = lax.dot_general(r, w_ref[...], (((1,), (1,)), ((), ())),
                        preferred_element_type=jnp.float32)
    o_ref[...] = jax.nn.sigmoid(y + b_ref[...])


_tc = pl.pallas_call(
    _tc_body,
    grid=(B // _BLK,),
    in_specs=[
        pl.BlockSpec((_BLK, V), lambda i: (i, 0)),
        pl.BlockSpec((V, D), lambda i: (0, 0)),
        pl.BlockSpec((V, D), lambda i: (0, 0)),
        pl.BlockSpec((1, V), lambda i: (0, 0)),
    ],
    out_specs=pl.BlockSpec((_BLK, V), lambda i: (i, 0)),
    out_shape=jax.ShapeDtypeStruct((B, V), jnp.float32),
)


def kernel(x, emb, W, b):
    counts = _hist()(x.astype(jnp.int32))
    return _tc(counts, emb, W, b.reshape(1, V))


# speedup vs baseline: 90.7794x; 1.3754x over previous
import functools

import jax
import jax.numpy as jnp
from jax import lax
from jax.experimental import pallas as pl
from jax.experimental.pallas import tpu as pltpu
from jax.experimental.pallas import tpu_sc as plsc

B = 16384
L = 200
V = 1000
D = 64

_NC, _NS = 2, 16
_NW = _NC * _NS
_S = B // _NW
_G = 64
_NCHUNK = _S // _G
_NGRP = L // 16


def _hist_body(x_hbm, c_hbm, idx_v, cnt_v):
    wid = lax.axis_index("s") * _NC + lax.axis_index("c")
    base = wid * _S

    ones = jnp.full((16,), 1.0, jnp.float32)
    zeros = jnp.full((16,), 0.0, jnp.float32)
    lane = lax.iota(jnp.int32, 16)
    tailmask = lane >= (16 - (L - _NGRP * 16))

    def _clear(k, _):
        def _clear_row(c, _r):
            cnt_v[k, pl.ds(c * 16, 16)] = zeros
            return _r
        lax.fori_loop(0, V // 16 + 1, _clear_row, None)
        return _
    lax.fori_loop(0, _G, _clear, None)

    def _chunk(k, _):
        row0 = base + k * _G
        pltpu.sync_copy(x_hbm.at[pl.ds(row0, _G)], idx_v)

        def _scatter(i, _c):
            rowv = jnp.full((16,), i, jnp.int32)
            idxs = [idx_v[i, pl.ds(j * 16, 16)] for j in range(_NGRP)]
            idxs.append(idx_v[i, pl.ds(L - 16, 16)])
            for j in range(_NGRP):
                plsc.addupdate_scatter(cnt_v, [rowv, idxs[j]], ones)
            plsc.addupdate_scatter(cnt_v, [rowv, idxs[_NGRP]], ones,
                                   mask=tailmask)
            return _c
        lax.fori_loop(0, _G, _scatter, None)

        pltpu.sync_copy(cnt_v, c_hbm.at[pl.ds(row0, _G)])

        def _rezero(i, _c):
            rowv = jnp.full((16,), i, jnp.int32)
            idxs = [idx_v[i, pl.ds(j * 16, 16)] for j in range(_NGRP)]
            idxs.append(idx_v[i, pl.ds(L - 16, 16)])
            for j in range(_NGRP + 1):
                plsc.store_scatter(cnt_v, [rowv, idxs[j]], zeros)
            return _c
        lax.fori_loop(0, _G, _rezero, None)
        return _
    lax.fori_loop(0, _NCHUNK, _chunk, None)


@functools.cache
def _hist():
    return functools.partial(
        pl.kernel,
        mesh=plsc.VectorSubcoreMesh(core_axis_name="c", subcore_axis_name="s"),
        out_type=jax.ShapeDtypeStruct((B, V), jnp.float32),
        scratch_types=[
            pltpu.VMEM((_G, L), jnp.int32),
            pltpu.VMEM((_G, V), jnp.float32),
        ],
        compiler_params=pltpu.CompilerParams(needs_layout_passes=False),
    )(_hist_body)


_BLK = 2048


def _tc_body(c_ref, emb_ref, w_ref, b_ref, o_ref):
    m = jnp.dot(c_ref[...], emb_ref[...], preferred_element_type=jnp.float32)
    r = jnp.maximum(m * (1.0 / L), 0.0)
    yt = lax.dot_general(w_ref[...], r, (((1,), (1,)), ((), ())),
                         preferred_element_type=jnp.float32)
    o_ref[...] = jax.nn.sigmoid(yt + b_ref[...])


_tc = pl.pallas_call(
    _tc_body,
    grid=(B // _BLK,),
    in_specs=[
        pl.BlockSpec((_BLK, V), lambda i: (i, 0)),
        pl.BlockSpec((V, D), lambda i: (0, 0)),
        pl.BlockSpec((V, D), lambda i: (0, 0)),
        pl.BlockSpec((V, 1), lambda i: (0, 0)),
    ],
    out_specs=pl.BlockSpec((V, _BLK), lambda i: (0, i)),
    out_shape=jax.ShapeDtypeStruct((V, B), jnp.float32),
)


def kernel(x, emb, W, b):
    counts = _hist()(x.astype(jnp.int32))
    yt = _tc(counts, emb, W, b.reshape(V, 1))
    return yt.T


# speedup vs baseline: 109.8999x; 1.2106x over previous
import functools

import jax
import jax.numpy as jnp
from jax import lax
from jax.experimental import pallas as pl
from jax.experimental.pallas import tpu as pltpu
from jax.experimental.pallas import tpu_sc as plsc

B = 16384
L = 200
V = 1000
D = 64

_NC, _NS = 2, 16
_NW = _NC * _NS
_S = B // _NW
_G = 32
_NCHUNK = _S // _G
_NGRP = L // 16
_AW = (_NGRP + 1) * 16


def _hist_body(x_hbm, c_hbm,
               idx0, idx1, adr0, adr1, cnt0, cnt1,
               si0, si1, so0, so1):
    wid = lax.axis_index("s") * _NC + lax.axis_index("c")
    base = wid * _S

    idx_b, adr_b, cnt_b = (idx0, idx1), (adr0, adr1), (cnt0, cnt1)
    si_b, so_b = (si0, si1), (so0, so1)

    ones = jnp.full((16,), 1.0, jnp.float32)
    zeros = jnp.full((16,), 0.0, jnp.float32)
    lane = lax.iota(jnp.int32, 16)
    tailmask = lane >= (16 - (L - _NGRP * 16))

    def _start_idx(k, b):
        row0 = base + (k % _NCHUNK) * _G
        pltpu.async_copy(x_hbm.at[pl.ds(row0, _G)], idx_b[b], si_b[b])

    def _wait_idx(b):
        pltpu.make_async_copy(x_hbm.at[pl.ds(base, _G)], idx_b[b],
                              si_b[b]).wait()

    def _start_out(k, b):
        row0 = base + k * _G
        pltpu.async_copy(cnt_b[b], c_hbm.at[pl.ds(row0, _G)], so_b[b])

    def _wait_out(b):
        pltpu.make_async_copy(cnt_b[b], c_hbm.at[pl.ds(base, _G)],
                              so_b[b]).wait()

    def _hist_chunk(b):
        cnt_v, idx_v, adr_v = cnt_b[b], idx_b[b], adr_b[b]

        def _scatter(i, _c):
            rowv = jnp.full((16,), i, jnp.int32)
            idxs = [idx_v[i, pl.ds(j * 16, 16)] for j in range(_NGRP)]
            idxs.append(idx_v[i, pl.ds(L - 16, 16)])
            for j in range(_NGRP):
                plsc.addupdate_scatter(cnt_v, [rowv, idxs[j]], ones)
            plsc.addupdate_scatter(cnt_v, [rowv, idxs[_NGRP]], ones,
                                   mask=tailmask)
            for j in range(_NGRP + 1):
                adr_v[i, pl.ds(j * 16, 16)] = idxs[j]
            return _c
        lax.fori_loop(0, _G, _scatter, None)

    def _rezero_chunk(b):
        cnt_v, adr_v = cnt_b[b], adr_b[b]

        def _rz(i, _c):
            rowv = jnp.full((16,), i, jnp.int32)
            idxs = [adr_v[i, pl.ds(j * 16, 16)] for j in range(_NGRP + 1)]
            for j in range(_NGRP + 1):
                plsc.store_scatter(cnt_v, [rowv, idxs[j]], zeros)
            return _c
        lax.fori_loop(0, _G, _rz, None)

    for b in (0, 1):
        def _clear(i, _, _b=b):
            def _clear_row(c, _r):
                cnt_b[_b][i, pl.ds(c * 16, 16)] = zeros
                return _r
            lax.fori_loop(0, V // 16 + 1, _clear_row, None)
            return _
        lax.fori_loop(0, _G, _clear, None)

    _start_idx(0, 0)
    _start_idx(1, 1)
    for b in (0, 1):
        _wait_idx(b)
        _hist_chunk(b)
        _start_idx(b + 2, b)
        _start_out(b, b)

    def _pair(h, _):
        for b in (0, 1):
            k = 2 * h + b
            _wait_out(b)
            _rezero_chunk(b)
            _wait_idx(b)
            _hist_chunk(b)
            _start_idx(k + 2, b)
            _start_out(k, b)
        return _
    lax.fori_loop(1, _NCHUNK // 2, _pair, None)

    for b in (0, 1):
        _wait_idx(b)
        _wait_out(b)


@functools.cache
def _hist():
    return functools.partial(
        pl.kernel,
        mesh=plsc.VectorSubcoreMesh(core_axis_name="c", subcore_axis_name="s"),
        out_type=jax.ShapeDtypeStruct((B, V), jnp.float32),
        scratch_types=[
            pltpu.VMEM((_G, L), jnp.int32),
            pltpu.VMEM((_G, L), jnp.int32),
            pltpu.VMEM((_G, _AW), jnp.int32),
            pltpu.VMEM((_G, _AW), jnp.int32),
            pltpu.VMEM((_G, V), jnp.float32),
            pltpu.VMEM((_G, V), jnp.float32),
            pltpu.SemaphoreType.DMA,
            pltpu.SemaphoreType.DMA,
            pltpu.SemaphoreType.DMA,
            pltpu.SemaphoreType.DMA,
        ],
        compiler_params=pltpu.CompilerParams(needs_layout_passes=False),
    )(_hist_body)


_BLK = 2048


def _tc_body(c_ref, emb_ref, w_ref, b_ref, o_ref):
    m = jnp.dot(c_ref[...], emb_ref[...], preferred_element_type=jnp.float32)
    r = jnp.maximum(m * (1.0 / L), 0.0)
    yt = lax.dot_general(w_ref[...], r, (((1,), (1,)), ((), ())),
                         preferred_element_type=jnp.float32)
    o_ref[...] = jax.nn.sigmoid(yt + b_ref[...])


_tc = pl.pallas_call(
    _tc_body,
    grid=(B // _BLK,),
    in_specs=[
        pl.BlockSpec((_BLK, V), lambda i: (i, 0)),
        pl.BlockSpec((V, D), lambda i: (0, 0)),
        pl.BlockSpec((V, D), lambda i: (0, 0)),
        pl.BlockSpec((V, 1), lambda i: (0, 0)),
    ],
    out_specs=pl.BlockSpec((V, _BLK), lambda i: (0, i)),
    out_shape=jax.ShapeDtypeStruct((V, B), jnp.float32),
)


def kernel(x, emb, W, b):
    counts = _hist()(x.astype(jnp.int32))
    yt = _tc(counts, emb, W, b.reshape(V, 1))
    return yt.T
